# baseline (speedup 1.0000x reference)
import jax
import jax.numpy as jnp
from jax.experimental import pallas as pl
from jax.experimental.pallas import tpu as pltpu

_VOCAB = 100000
_D = 128
_BATCH = 16384
_BLK = 512
_NB = _BATCH // _BLK
_BROWS = (_VOCAB + 127) // 128
_EXT = (_NB + 1) * _BLK


def _k1(i1s, l1c, w1_hbm, b1t, e1_out, pb_out, tbl, bb, sem):
    p = pl.program_id(0)
    b = pl.program_id(1)

    @pl.when((p == 0) & (b == 0))
    def _():
        pltpu.make_async_copy(w1_hbm, tbl, sem).start()

    @pl.when(p == 0)
    def _():
        for mi in range(_BLK):
            v = i1s[0, 0, mi]
            bb[mi : mi + 1, :] = b1t[v >> 7]
        lane = jax.lax.broadcasted_iota(jnp.int32, (_BLK, _D), 1)
        pb_out[:] = jnp.sum(
            jnp.where(lane == l1c[:], bb[:], 0.0), axis=1, keepdims=True
        )

    @pl.when((p == 1) & (b == 0))
    def _():
        pltpu.make_async_copy(w1_hbm, tbl, sem).wait()

    @pl.when(p == 1)
    def _():
        for mi in range(_BLK):
            v = i1s[0, 0, mi]
            e1_out[mi : mi + 1, :] = tbl[v]


def _k2(i2s, l2c, e1blk, pb1blk, pb2in, w2_hbm, b2t, out, pb2o, tbl, bb, sem):
    p = pl.program_id(0)
    b = pl.program_id(1)

    @pl.when((p == 0) & (b == 0))
    def _():
        pltpu.make_async_copy(w2_hbm, tbl, sem).start()

    @pl.when(p == 0)
    def _():
        for mi in range(_BLK):
            v = i2s[0, 0, mi]
            bb[mi : mi + 1, :] = b2t[v >> 7]
        lane = jax.lax.broadcasted_iota(jnp.int32, (_BLK, _D), 1)
        pb2o[:] = jnp.sum(
            jnp.where(lane == l2c[:], bb[:], 0.0), axis=1, keepdims=True
        )

    @pl.when((p == 1) & (b == 0))
    def _():
        pltpu.make_async_copy(w2_hbm, tbl, sem).wait()

    @pl.when(p == 1)
    def _():
        for mi in range(_BLK):
            v = i2s[0, 0, mi]
            bb[mi : mi + 1, :] = tbl[v]
        out[:] = (
            jnp.sum(e1blk[:] * bb[:], axis=1, keepdims=True)
            + pb1blk[:]
            + pb2in[:]
        )


def kernel(i1, i2, W1, W2, b1, b2):
    w1r = W1.reshape(_VOCAB, 1, _D)
    w2r = W2.reshape(_VOCAB, 1, _D)
    pad = _BROWS * 128 - _VOCAB
    b1t = jnp.pad(b1[:, 0], (0, pad)).reshape(_BROWS, 1, 128)
    b2t = jnp.pad(b2[:, 0], (0, pad)).reshape(_BROWS, 1, 128)
    i1m = i1.reshape(_NB, 1, _BLK)
    i2m = i2.reshape(_NB, 1, _BLK)
    l1c = (i1 & 127).reshape(_BATCH, 1)
    l2c = (i2 & 127).reshape(_BATCH, 1)

    cp = pltpu.CompilerParams(
        dimension_semantics=("arbitrary", "arbitrary"),
        vmem_limit_bytes=64 * 1024 * 1024,
    )
    smem_spec = pl.BlockSpec(
        (1, 1, _BLK), lambda p, b: (b, 0, 0), memory_space=pltpu.SMEM
    )
    lcol_spec = pl.BlockSpec((_BLK, 1), lambda p, b: (b, 0))
    btab_spec = pl.BlockSpec((_BROWS, 1, 128), lambda p, b: (0, 0, 0))
    ph1_rows = pl.BlockSpec(
        (_BLK, _D), lambda p, b: (jnp.where(p == 1, b, _NB), 0)
    )
    ph0_col = pl.BlockSpec(
        (_BLK, 1), lambda p, b: (jnp.where(p == 0, b, _NB), 0)
    )
    ph1_col = pl.BlockSpec(
        (_BLK, 1), lambda p, b: (jnp.where(p == 1, b, _NB), 0)
    )

    e1, pb1 = pl.pallas_call(
        _k1,
        grid=(2, _NB),
        in_specs=[
            smem_spec,
            lcol_spec,
            pl.BlockSpec(memory_space=pl.ANY),
            btab_spec,
        ],
        out_specs=[ph1_rows, ph0_col],
        out_shape=[
            jax.ShapeDtypeStruct((_EXT, _D), jnp.float32),
            jax.ShapeDtypeStruct((_EXT, 1), jnp.float32),
        ],
        scratch_shapes=[
            pltpu.VMEM((_VOCAB, 1, _D), jnp.float32),
            pltpu.VMEM((_BLK, _D), jnp.float32),
            pltpu.SemaphoreType.DMA,
        ],
        compiler_params=cp,
    )(i1m, l1c, w1r, b1t)

    pb2buf = jnp.zeros((_EXT, 1), jnp.float32)
    outm, _pb2 = pl.pallas_call(
        _k2,
        grid=(2, _NB),
        in_specs=[
            smem_spec,
            lcol_spec,
            ph1_rows,
            ph1_col,
            ph1_col,
            pl.BlockSpec(memory_space=pl.ANY),
            btab_spec,
        ],
        out_specs=[ph1_col, ph0_col],
        out_shape=[
            jax.ShapeDtypeStruct((_EXT, 1), jnp.float32),
            jax.ShapeDtypeStruct((_EXT, 1), jnp.float32),
        ],
        input_output_aliases={4: 1},
        scratch_shapes=[
            pltpu.VMEM((_VOCAB, 1, _D), jnp.float32),
            pltpu.VMEM((_BLK, _D), jnp.float32),
            pltpu.SemaphoreType.DMA,
        ],
        compiler_params=cp,
    )(i2m, l2c, e1, pb1, pb2buf, w2r, b2t)
    return outm[:_BATCH]


# speedup vs baseline: 1.3239x; 1.3239x over previous
import jax
import jax.numpy as jnp
from jax.experimental import pallas as pl
from jax.experimental.pallas import tpu as pltpu

_VOCAB = 100000
_D = 128
_BATCH = 16384
_BLK = 2048
_NB = _BATCH // _BLK
_BROWS = (_VOCAB + 127) // 128
_EXT = (_NB + 1) * _BLK


def _k1(i1s, l1c, w1_hbm, b1t, e1_out, pb_out, tbl, bb, sem):
    p = pl.program_id(0)
    b = pl.program_id(1)

    @pl.when((p == 0) & (b == 0))
    def _():
        pltpu.make_async_copy(w1_hbm, tbl, sem).start()

    @pl.when(p == 0)
    def _():
        for mi in range(_BLK):
            v = i1s[0, 0, mi]
            bb[mi : mi + 1, :] = b1t[v >> 7]
        lane = jax.lax.broadcasted_iota(jnp.int32, (_BLK, _D), 1)
        pb_out[:] = jnp.sum(
            jnp.where(lane == l1c[:], bb[:], 0.0), axis=1, keepdims=True
        )

    @pl.when((p == 1) & (b == 0))
    def _():
        pltpu.make_async_copy(w1_hbm, tbl, sem).wait()

    @pl.when(p == 1)
    def _():
        for mi in range(_BLK):
            v = i1s[0, 0, mi]
            e1_out[mi : mi + 1, :] = tbl[v]


def _k2(i2s, l2c, e1blk, pb1blk, pb2in, w2_hbm, b2t, out, pb2o, tbl, bb, sem):
    p = pl.program_id(0)
    b = pl.program_id(1)

    @pl.when((p == 0) & (b == 0))
    def _():
        pltpu.make_async_copy(w2_hbm, tbl, sem).start()

    @pl.when(p == 0)
    def _():
        for mi in range(_BLK):
            v = i2s[0, 0, mi]
            bb[mi : mi + 1, :] = b2t[v >> 7]
        lane = jax.lax.broadcasted_iota(jnp.int32, (_BLK, _D), 1)
        pb2o[:] = jnp.sum(
            jnp.where(lane == l2c[:], bb[:], 0.0), axis=1, keepdims=True
        )

    @pl.when((p == 1) & (b == 0))
    def _():
        pltpu.make_async_copy(w2_hbm, tbl, sem).wait()

    @pl.when(p == 1)
    def _():
        for mi in range(_BLK):
            v = i2s[0, 0, mi]
            bb[mi : mi + 1, :] = tbl[v]
        out[:] = (
            jnp.sum(e1blk[:] * bb[:], axis=1, keepdims=True)
            + pb1blk[:]
            + pb2in[:]
        )


def kernel(i1, i2, W1, W2, b1, b2):
    w1r = W1.reshape(_VOCAB, 1, _D)
    w2r = W2.reshape(_VOCAB, 1, _D)
    pad = _BROWS * 128 - _VOCAB
    b1t = jnp.pad(b1[:, 0], (0, pad)).reshape(_BROWS, 1, 128)
    b2t = jnp.pad(b2[:, 0], (0, pad)).reshape(_BROWS, 1, 128)
    i1m = i1.reshape(_NB, 1, _BLK)
    i2m = i2.reshape(_NB, 1, _BLK)
    l1c = (i1 & 127).reshape(_BATCH, 1)
    l2c = (i2 & 127).reshape(_BATCH, 1)

    cp = pltpu.CompilerParams(
        dimension_semantics=("arbitrary", "arbitrary"),
        vmem_limit_bytes=64 * 1024 * 1024,
    )
    smem_spec = pl.BlockSpec(
        (1, 1, _BLK), lambda p, b: (b, 0, 0), memory_space=pltpu.SMEM
    )
    lcol_spec = pl.BlockSpec((_BLK, 1), lambda p, b: (b, 0))
    btab_spec = pl.BlockSpec((_BROWS, 1, 128), lambda p, b: (0, 0, 0))
    ph1_rows = pl.BlockSpec(
        (_BLK, _D), lambda p, b: (jnp.where(p == 1, b, _NB), 0)
    )
    ph0_col = pl.BlockSpec(
        (_BLK, 1), lambda p, b: (jnp.where(p == 0, b, _NB), 0)
    )
    ph1_col = pl.BlockSpec(
        (_BLK, 1), lambda p, b: (jnp.where(p == 1, b, _NB), 0)
    )

    e1, pb1 = pl.pallas_call(
        _k1,
        grid=(2, _NB),
        in_specs=[
            smem_spec,
            lcol_spec,
            pl.BlockSpec(memory_space=pl.ANY),
            btab_spec,
        ],
        out_specs=[ph1_rows, ph0_col],
        out_shape=[
            jax.ShapeDtypeStruct((_EXT, _D), jnp.float32),
            jax.ShapeDtypeStruct((_EXT, 1), jnp.float32),
        ],
        scratch_shapes=[
            pltpu.VMEM((_VOCAB, 1, _D), jnp.float32),
            pltpu.VMEM((_BLK, _D), jnp.float32),
            pltpu.SemaphoreType.DMA,
        ],
        compiler_params=cp,
    )(i1m, l1c, w1r, b1t)

    pb2buf = jnp.zeros((_EXT, 1), jnp.float32)
    outm, _pb2 = pl.pallas_call(
        _k2,
        grid=(2, _NB),
        in_specs=[
            smem_spec,
            lcol_spec,
            ph1_rows,
            ph1_col,
            ph1_col,
            pl.BlockSpec(memory_space=pl.ANY),
            btab_spec,
        ],
        out_specs=[ph1_col, ph0_col],
        out_shape=[
            jax.ShapeDtypeStruct((_EXT, 1), jnp.float32),
            jax.ShapeDtypeStruct((_EXT, 1), jnp.float32),
        ],
        input_output_aliases={4: 1},
        scratch_shapes=[
            pltpu.VMEM((_VOCAB, 1, _D), jnp.float32),
            pltpu.VMEM((_BLK, _D), jnp.float32),
            pltpu.SemaphoreType.DMA,
        ],
        compiler_params=cp,
    )(i2m, l2c, e1, pb1, pb2buf, w2r, b2t)
    return outm[:_BATCH]


# speedup vs baseline: 1.7440x; 1.3173x over previous
import jax
import jax.numpy as jnp
from jax.experimental import pallas as pl
from jax.experimental.pallas import tpu as pltpu

_VOCAB = 100000
_D = 128
_BATCH = 16384
_BLK = 2048
_NB = _BATCH // _BLK
_BROWS = (_VOCAB + 127) // 128
_EXT = (_NB + 1) * _BLK


def _k1(i1s, w1_hbm, b1t, e1_out, pbr1_out, tbl, sem):
    p = pl.program_id(0)
    b = pl.program_id(1)
    lane = jax.lax.broadcasted_iota(jnp.int32, (1, _D), 1)

    @pl.when((p == 0) & (b == 0))
    def _():
        pltpu.make_async_copy(w1_hbm, tbl, sem).start()

    @pl.when(p == 0)
    def _():
        for mi in range(_BLK):
            v = i1s[0, 0, mi]
            pbr1_out[mi : mi + 1, :] = jnp.where(
                lane == (v & 127), b1t[v >> 7], 0.0
            )

    @pl.when((p == 1) & (b == 0))
    def _():
        pltpu.make_async_copy(w1_hbm, tbl, sem).wait()

    @pl.when(p == 1)
    def _():
        for mi in range(_BLK):
            v = i1s[0, 0, mi]
            e1_out[mi : mi + 1, :] = tbl[v]


def _k2(i2s, e1blk, pbr1blk, pbr2in, w2_hbm, b2t, out, pbr2o, tbl, e2, sem):
    p = pl.program_id(0)
    b = pl.program_id(1)
    lane = jax.lax.broadcasted_iota(jnp.int32, (1, _D), 1)

    @pl.when((p == 0) & (b == 0))
    def _():
        pltpu.make_async_copy(w2_hbm, tbl, sem).start()

    @pl.when(p == 0)
    def _():
        for mi in range(_BLK):
            v = i2s[0, 0, mi]
            pbr2o[mi : mi + 1, :] = jnp.where(
                lane == (v & 127), b2t[v >> 7], 0.0
            )

    @pl.when((p == 1) & (b == 0))
    def _():
        pltpu.make_async_copy(w2_hbm, tbl, sem).wait()

    @pl.when(p == 1)
    def _():
        for mi in range(_BLK):
            v = i2s[0, 0, mi]
            e2[mi : mi + 1, :] = tbl[v]
        out[:] = jnp.sum(
            e1blk[:] * e2[:] + pbr1blk[:] + pbr2in[:],
            axis=1,
            keepdims=True,
        )


def kernel(i1, i2, W1, W2, b1, b2):
    w1r = W1.reshape(_VOCAB, 1, _D)
    w2r = W2.reshape(_VOCAB, 1, _D)
    pad = _BROWS * 128 - _VOCAB
    b1t = jnp.pad(b1[:, 0], (0, pad)).reshape(_BROWS, 1, 128)
    b2t = jnp.pad(b2[:, 0], (0, pad)).reshape(_BROWS, 1, 128)
    i1m = i1.reshape(_NB, 1, _BLK)
    i2m = i2.reshape(_NB, 1, _BLK)

    cp = pltpu.CompilerParams(
        dimension_semantics=("arbitrary", "arbitrary"),
        vmem_limit_bytes=64 * 1024 * 1024,
    )
    smem_spec = pl.BlockSpec(
        (1, 1, _BLK), lambda p, b: (b, 0, 0), memory_space=pltpu.SMEM
    )
    btab_spec = pl.BlockSpec((_BROWS, 1, 128), lambda p, b: (0, 0, 0))
    ph0_rows = pl.BlockSpec(
        (_BLK, _D), lambda p, b: (jnp.where(p == 0, b, _NB), 0)
    )
    ph1_rows = pl.BlockSpec(
        (_BLK, _D), lambda p, b: (jnp.where(p == 1, b, _NB), 0)
    )
    ph1_col = pl.BlockSpec(
        (_BLK, 1), lambda p, b: (jnp.where(p == 1, b, _NB), 0)
    )

    e1, pbr1 = pl.pallas_call(
        _k1,
        grid=(2, _NB),
        in_specs=[
            smem_spec,
            pl.BlockSpec(memory_space=pl.ANY),
            btab_spec,
        ],
        out_specs=[ph1_rows, ph0_rows],
        out_shape=[
            jax.ShapeDtypeStruct((_EXT, _D), jnp.float32),
            jax.ShapeDtypeStruct((_EXT, _D), jnp.float32),
        ],
        scratch_shapes=[
            pltpu.VMEM((_VOCAB, 1, _D), jnp.float32),
            pltpu.SemaphoreType.DMA,
        ],
        compiler_params=cp,
    )(i1m, w1r, b1t)

    pbr2buf = jnp.zeros((_EXT, _D), jnp.float32)
    outm, _pbr2 = pl.pallas_call(
        _k2,
        grid=(2, _NB),
        in_specs=[
            smem_spec,
            ph1_rows,
            ph1_rows,
            ph1_rows,
            pl.BlockSpec(memory_space=pl.ANY),
            btab_spec,
        ],
        out_specs=[ph1_col, ph0_rows],
        out_shape=[
            jax.ShapeDtypeStruct((_EXT, 1), jnp.float32),
            jax.ShapeDtypeStruct((_EXT, _D), jnp.float32),
        ],
        input_output_aliases={3: 1},
        scratch_shapes=[
            pltpu.VMEM((_VOCAB, 1, _D), jnp.float32),
            pltpu.VMEM((_BLK, _D), jnp.float32),
            pltpu.SemaphoreType.DMA,
        ],
        compiler_params=cp,
    )(i2m, e1, pbr1, pbr2buf, w2r, b2t)
    return outm[:_BATCH]
